# Initial kernel scaffold; baseline (speedup 1.0000x reference)
import jax
import jax.numpy as jnp
from jax.experimental import pallas as pl


def kernel(data, p_aux):
    raise NotImplementedError("write your pallas kernel here")


# speedup vs baseline: 1.9874x; 1.9874x over previous
import functools

import jax
import jax.numpy as jnp
from jax import lax
from jax.experimental import pallas as pl
from jax.experimental.pallas import tpu as pltpu
from jax.experimental.pallas import tpu_sc as plsc

_N = 26
_K = 100000
_B = 16384
_NC = 2
_NS = 16
_LANES = 16
_CHUNK = 8192


def _sc_gather_body(p_hbm, idx_hbm, out_hbm, row_v, idx_v, g_v):
    wid = lax.axis_index("s") * _NC + lax.axis_index("c")

    @pl.when(wid < _N)
    def _():
        pltpu.sync_copy(p_hbm.at[wid], row_v)

        def chunk_body(ci, carry):
            base = ci * _CHUNK
            pltpu.sync_copy(idx_hbm.at[wid, pl.ds(base, _CHUNK)], idx_v)

            def lane_body(i, c2):
                sl = pl.ds(i * _LANES, _LANES)
                g_v[sl] = plsc.load_gather(row_v, [idx_v[sl]])
                return c2

            lax.fori_loop(0, _CHUNK // _LANES, lane_body, 0, unroll=4)
            pltpu.sync_copy(g_v, out_hbm.at[wid, pl.ds(base, _CHUNK)])
            return carry

        lax.fori_loop(0, _B // _CHUNK, chunk_body, 0)


_sc_gather = functools.partial(
    pl.kernel,
    out_type=jax.ShapeDtypeStruct((_N, _B), jnp.float32),
    mesh=plsc.VectorSubcoreMesh(core_axis_name="c", subcore_axis_name="s"),
    compiler_params=pltpu.CompilerParams(needs_layout_passes=False),
    scratch_types=[
        pltpu.VMEM((_K,), jnp.float32),
        pltpu.VMEM((_CHUNK,), jnp.int32),
        pltpu.VMEM((_CHUNK,), jnp.float32),
    ],
)(_sc_gather_body)


def _tc_finish_body(p_ref, g_ref, out_ref):
    p = p_ref[...]
    s = jnp.sum(jax.nn.sigmoid(p), axis=1, keepdims=True)
    g = g_ref[...]
    out_ref[...] = jnp.log(jax.nn.sigmoid(g) / s)


def kernel(data, p_aux):
    idx_t = data.T
    g_t = _sc_gather(p_aux, idx_t)
    out_t = pl.pallas_call(
        _tc_finish_body,
        out_shape=jax.ShapeDtypeStruct((_N, _B), jnp.float32),
    )(p_aux, g_t)
    return out_t.T


# speedup vs baseline: 2.7453x; 1.3813x over previous
import functools

import jax
import jax.numpy as jnp
from jax import lax
from jax.experimental import pallas as pl
from jax.experimental.pallas import tpu as pltpu
from jax.experimental.pallas import tpu_sc as plsc

_N = 26
_K = 100000
_B = 16384
_NC = 2
_NS = 16
_LANES = 16
_CH = 4096
_NCH = _B // _CH
_CK = 8192
_NBK = -(-_K // _CK)


def _sc_gather_body(p_hbm, idx_hbm, out_hbm, row_v, i0, i1, i2, i3, g0, g1,
                    sem_row, sem_idx, sem_o0, sem_o1):
    wid = lax.axis_index("s") * _NC + lax.axis_index("c")

    @pl.when(wid < _N)
    def _():
        idx_bufs = (i0, i1, i2, i3)
        g_bufs = (g0, g1)
        osems = (sem_o0, sem_o1)

        row_cp = pltpu.async_copy(p_hbm.at[wid], row_v, sem_row)
        icps = [
            pltpu.async_copy(
                idx_hbm.at[wid, pl.ds(c * _CH, _CH)], idx_bufs[c], sem_idx)
            for c in range(_NCH)
        ]
        row_cp.wait()

        ocps = [None, None]
        for c in range(_NCH):
            b = c % 2
            icps[c].wait()
            if ocps[b] is not None:
                ocps[b].wait()

            def _gather(ib, gb):
                @plsc.parallel_loop(0, _CH, _LANES, unroll=8)
                def _g(i):
                    sl = pl.ds(i, _LANES)
                    gb[sl] = plsc.load_gather(row_v, [ib[sl]])

            _gather(idx_bufs[c], g_bufs[b])
            ocps[b] = pltpu.async_copy(
                g_bufs[b], out_hbm.at[wid, pl.ds(c * _CH, _CH)], osems[b])
        ocps[0].wait()
        ocps[1].wait()


_sc_gather = functools.partial(
    pl.kernel,
    out_type=jax.ShapeDtypeStruct((_N, _B), jnp.float32),
    mesh=plsc.VectorSubcoreMesh(core_axis_name="c", subcore_axis_name="s"),
    compiler_params=pltpu.CompilerParams(needs_layout_passes=False),
    scratch_types=[
        pltpu.VMEM((_K,), jnp.float32),
        pltpu.VMEM((_CH,), jnp.int32),
        pltpu.VMEM((_CH,), jnp.int32),
        pltpu.VMEM((_CH,), jnp.int32),
        pltpu.VMEM((_CH,), jnp.int32),
        pltpu.VMEM((_CH,), jnp.float32),
        pltpu.VMEM((_CH,), jnp.float32),
        pltpu.SemaphoreType.DMA,
        pltpu.SemaphoreType.DMA,
        pltpu.SemaphoreType.DMA,
        pltpu.SemaphoreType.DMA,
    ],
)(_sc_gather_body)


def _tc_sum_body(p_ref, s_ref):
    i = pl.program_id(0)

    @pl.when(i == 0)
    def _():
        s_ref[...] = jnp.zeros_like(s_ref)

    x = p_ref[...]
    col = i * _CK + lax.broadcasted_iota(jnp.int32, x.shape, 1)
    sig = jnp.where(col < _K, jax.nn.sigmoid(x), 0.0)
    part = jnp.sum(sig, axis=1, keepdims=True)
    s_ref[...] += jnp.broadcast_to(part, s_ref.shape)


def _tc_finish_body(g_ref, s_ref, o_ref):
    s = s_ref[:, 0:1]
    o_ref[...] = jnp.log(jax.nn.sigmoid(g_ref[...]) / s)


def kernel(data, p_aux):
    idx_t = data.T
    s = pl.pallas_call(
        _tc_sum_body,
        grid=(_NBK,),
        in_specs=[pl.BlockSpec((_N, _CK), lambda i: (0, i))],
        out_specs=pl.BlockSpec((_N, 128), lambda i: (0, 0)),
        out_shape=jax.ShapeDtypeStruct((_N, 128), jnp.float32),
    )(p_aux)
    g_t = _sc_gather(p_aux, idx_t)
    out_t = pl.pallas_call(
        _tc_finish_body,
        out_shape=jax.ShapeDtypeStruct((_N, _B), jnp.float32),
    )(g_t, s)
    return out_t.T
